# Initial kernel scaffold; baseline (speedup 1.0000x reference)
import jax
import jax.numpy as jnp
from jax.experimental import pallas as pl


def kernel(class_pred, class_gt):
    raise NotImplementedError("write your pallas kernel here")


# speedup vs baseline: 21.7362x; 21.7362x over previous
import jax
import jax.numpy as jnp
from jax.experimental import pallas as pl
from jax.experimental.pallas import tpu as pltpu

_N_CLASS = 21
_EPS = 1e-07
_LANES = 128
_SUB = 8
_STEPS = 16


def _fwiou_kernel(p_ref, g_ref, o_ref, tacc_ref, iacc_ref, *, n):
    j = pl.program_id(0)

    @pl.when(j == 0)
    def _():
        tacc_ref[...] = jnp.zeros_like(tacc_ref)
        iacc_ref[...] = jnp.zeros_like(iacc_ref)

    p = p_ref[...]
    g = g_ref[...]

    for c in range(_N_CLASS):
        pc = p == c
        gc = g == c
        a = jnp.where(pc, 1, 0)
        b = jnp.where(gc, 1, 0)
        i = jnp.where(pc & gc, 1, 0)
        tacc_ref[c] += jnp.sum(a + b, axis=0)
        iacc_ref[c] += jnp.sum(i, axis=0)

    @pl.when(j == _STEPS - 1)
    def _():
        t3 = tacc_ref[...].astype(jnp.float32)
        i3 = iacc_ref[...].astype(jnp.float32)
        tl = jnp.sum(jnp.sum(t3, axis=2), axis=1)
        il = jnp.sum(jnp.sum(i3, axis=2), axis=1)
        weight = tl * (1.0 / (2.0 * n))
        iou = (il + _EPS) / ((tl - il) + _EPS)
        r = jnp.sum(weight * iou)
        o_ref[...] = jnp.full((1, _LANES), r, jnp.float32)


def kernel(class_pred, class_gt):
    import functools

    n = class_pred.size
    rows = n // (_SUB * _LANES)
    block_rows = rows // _STEPS
    p = class_pred.reshape(rows, _SUB, _LANES)
    g = class_gt.reshape(rows, _SUB, _LANES)

    in_spec = pl.BlockSpec((block_rows, _SUB, _LANES), lambda j: (j, 0, 0))
    out = pl.pallas_call(
        functools.partial(_fwiou_kernel, n=float(n)),
        grid=(_STEPS,),
        in_specs=[in_spec, in_spec],
        out_specs=pl.BlockSpec((1, _LANES), lambda j: (0, 0)),
        out_shape=jax.ShapeDtypeStruct((1, _LANES), jnp.float32),
        scratch_shapes=[
            pltpu.VMEM((_N_CLASS, _SUB, _LANES), jnp.int32),
            pltpu.VMEM((_N_CLASS, _SUB, _LANES), jnp.int32),
        ],
        compiler_params=pltpu.CompilerParams(
            dimension_semantics=("arbitrary",),
        ),
        name="fwiou",
    )(p, g)
    return out[0, 0]


# speedup vs baseline: 64.5366x; 2.9691x over previous
import functools

import jax
import jax.numpy as jnp
from jax.experimental import pallas as pl
from jax.experimental.pallas import tpu as pltpu

_N_CLASS = 21
_EPS = 1e-07
_LANES = 128
_SUB = 8
_STEPS = 16
_DT = 15
_DI = 14


def _feed_single(planes, pend, w, k):
    while pend[k] is not None:
        a = pend[k]
        pend[k] = None
        stored = planes[k]
        if stored is None:
            s = a ^ w
            c = a & w
        else:
            u = a ^ w
            s = u ^ stored
            c = (a & w) | (u & stored)
        planes[k] = s
        w = c
        k += 1
    pend[k] = w


def _flush(planes, pend):
    for k in range(len(pend)):
        w = pend[k]
        if w is None:
            continue
        pend[k] = None
        kk = k
        while w is not None and kk < len(planes):
            stored = planes[kk]
            if stored is None:
                planes[kk] = w
                w = None
            else:
                planes[kk] = stored ^ w
                w = stored & w
                kk += 1


def _fwiou_kernel(p_ref, g_ref, o_ref, tpl_ref, ipl_ref, *, n, block_rows):
    j = pl.program_id(0)

    @pl.when(j == 0)
    def _():
        tpl_ref[...] = jnp.zeros_like(tpl_ref)
        ipl_ref[...] = jnp.zeros_like(ipl_ref)

    tplanes = [tpl_ref[k] for k in range(_DT)]
    iplanes = [ipl_ref[k] for k in range(_DI)]
    tpend = [None] * (_DT + 2)
    ipend = [None] * (_DI + 2)

    for i in range(block_rows):
        ohp = 1 << p_ref[i]
        ohg = 1 << g_ref[i]
        u = ohp ^ ohg
        both = ohp & ohg
        pl0 = tplanes[0]
        tplanes[0] = u ^ pl0
        _feed_single(tplanes, tpend, both | (u & pl0), 1)
        _feed_single(iplanes, ipend, both, 0)

    _flush(tplanes, tpend)
    _flush(iplanes, ipend)

    for k in range(_DT):
        tpl_ref[k] = tplanes[k]
    for k in range(_DI):
        ipl_ref[k] = iplanes[k]

    @pl.when(j == _STEPS - 1)
    def _():
        def class_counts(ref, depth):
            accs = []
            for c in range(_N_CLASS):
                acc = None
                for k in range(depth):
                    b = ((ref[k] >> c) & 1) << k
                    acc = b if acc is None else acc + b
                accs.append(acc)
            x = jnp.stack(accs).astype(jnp.float32)
            return jnp.sum(jnp.sum(x, axis=2), axis=1)

        tl = class_counts(tpl_ref, _DT)
        il = class_counts(ipl_ref, _DI)
        weight = tl * (1.0 / (2.0 * n))
        iou = (il + _EPS) / ((tl - il) + _EPS)
        r = jnp.sum(weight * iou)
        o_ref[...] = jnp.full((1, _LANES), r, jnp.float32)


def kernel(class_pred, class_gt):
    n = class_pred.size
    rows = n // (_SUB * _LANES)
    block_rows = rows // _STEPS
    p = class_pred.reshape(rows, _SUB, _LANES)
    g = class_gt.reshape(rows, _SUB, _LANES)

    in_spec = pl.BlockSpec((block_rows, _SUB, _LANES), lambda j: (j, 0, 0))
    out = pl.pallas_call(
        functools.partial(_fwiou_kernel, n=float(n), block_rows=block_rows),
        grid=(_STEPS,),
        in_specs=[in_spec, in_spec],
        out_specs=pl.BlockSpec((1, _LANES), lambda j: (0, 0)),
        out_shape=jax.ShapeDtypeStruct((1, _LANES), jnp.float32),
        scratch_shapes=[
            pltpu.VMEM((_DT, _SUB, _LANES), jnp.int32),
            pltpu.VMEM((_DI, _SUB, _LANES), jnp.int32),
        ],
        compiler_params=pltpu.CompilerParams(
            dimension_semantics=("arbitrary",),
        ),
        name="fwiou_csa",
    )(p, g)
    return out[0, 0]


# speedup vs baseline: 66.8136x; 1.0353x over previous
import functools

import jax
import jax.numpy as jnp
from jax.experimental import pallas as pl
from jax.experimental.pallas import tpu as pltpu

_N_CLASS = 21
_EPS = 1e-07
_LANES = 128
_SUB = 8
_STEPS = 16
_CHUNK = 64
_DT = 15
_DI = 14


def _feed_single(planes, pend, w, k):
    while pend[k] is not None:
        a = pend[k]
        pend[k] = None
        stored = planes[k]
        if stored is None:
            s = a ^ w
            c = a & w
        else:
            u = a ^ w
            s = u ^ stored
            c = (a & w) | (u & stored)
        planes[k] = s
        w = c
        k += 1
    pend[k] = w


def _merge_into(ref, depth, planes, pend):
    carry = None
    for kk in range(depth):
        local = None
        if kk < len(planes) and planes[kk] is not None:
            local = planes[kk]
        if kk < len(pend) and pend[kk] is not None:
            assert local is None
            local = pend[kk]
        if local is None and carry is None:
            continue
        old = ref[kk]
        if local is not None and carry is not None:
            u = local ^ carry
            s = u ^ old
            c = (local & carry) | (u & old)
        else:
            w = local if local is not None else carry
            s = old ^ w
            c = old & w
        ref[kk] = s
        carry = c if kk < depth - 1 else None


def _fwiou_kernel(p_ref, g_ref, o_ref, tpl_ref, ipl_ref, *, n, block_rows):
    j = pl.program_id(0)

    @pl.when(j == 0)
    def _():
        tpl_ref[...] = jnp.zeros_like(tpl_ref)
        ipl_ref[...] = jnp.zeros_like(ipl_ref)

    def chunk_body(ci, _):
        base = ci * _CHUNK
        p = p_ref[pl.ds(base, _CHUNK)]
        g = g_ref[pl.ds(base, _CHUNK)]
        ltp = [None] * 8
        lti = [None] * 8
        ltp_pend = [None] * 10
        lti_pend = [None] * 10
        for i2 in range(_CHUNK):
            ohp = 1 << p[i2]
            ohg = 1 << g[i2]
            u = ohp ^ ohg
            both = ohp & ohg
            pl0 = ltp[0]
            if pl0 is None:
                ltp[0] = u
                carry = both
            else:
                ltp[0] = u ^ pl0
                carry = both | (u & pl0)
            _feed_single(ltp, ltp_pend, carry, 1)
            _feed_single(lti, lti_pend, both, 0)
        _merge_into(tpl_ref, _DT, ltp, ltp_pend)
        _merge_into(ipl_ref, _DI, lti, lti_pend)
        return 0

    jax.lax.fori_loop(0, block_rows // _CHUNK, chunk_body, 0)

    @pl.when(j == _STEPS - 1)
    def _():
        def class_counts(ref, depth):
            accs = []
            for c in range(_N_CLASS):
                acc = None
                for k in range(depth):
                    b = ((ref[k] >> c) & 1) << k
                    acc = b if acc is None else acc + b
                accs.append(acc)
            x = jnp.stack(accs).astype(jnp.float32)
            return jnp.sum(jnp.sum(x, axis=2), axis=1)

        tl = class_counts(tpl_ref, _DT)
        il = class_counts(ipl_ref, _DI)
        weight = tl * (1.0 / (2.0 * n))
        iou = (il + _EPS) / ((tl - il) + _EPS)
        r = jnp.sum(weight * iou)
        o_ref[...] = jnp.full((1, _LANES), r, jnp.float32)


def kernel(class_pred, class_gt):
    n = class_pred.size
    rows = n // (_SUB * _LANES)
    block_rows = rows // _STEPS
    p = class_pred.reshape(rows, _SUB, _LANES)
    g = class_gt.reshape(rows, _SUB, _LANES)

    in_spec = pl.BlockSpec((block_rows, _SUB, _LANES), lambda j: (j, 0, 0))
    out = pl.pallas_call(
        functools.partial(_fwiou_kernel, n=float(n), block_rows=block_rows),
        grid=(_STEPS,),
        in_specs=[in_spec, in_spec],
        out_specs=pl.BlockSpec((1, _LANES), lambda j: (0, 0)),
        out_shape=jax.ShapeDtypeStruct((1, _LANES), jnp.float32),
        scratch_shapes=[
            pltpu.VMEM((_DT, _SUB, _LANES), jnp.int32),
            pltpu.VMEM((_DI, _SUB, _LANES), jnp.int32),
        ],
        compiler_params=pltpu.CompilerParams(
            dimension_semantics=("arbitrary",),
        ),
        name="fwiou_csa",
    )(p, g)
    return out[0, 0]


# speedup vs baseline: 66.8945x; 1.0012x over previous
import functools

import jax
import jax.numpy as jnp
from jax.experimental import pallas as pl
from jax.experimental.pallas import tpu as pltpu

_N_CLASS = 21
_EPS = 1e-07
_LANES = 128
_SUB = 8
_STEPS = 16
_NSETS = 2
_SUBCHUNK = 64
_DT = 14
_DI = 13


def _feed_single(planes, pend, w, k):
    while pend[k] is not None:
        a = pend[k]
        pend[k] = None
        stored = planes[k]
        if stored is None:
            s = a ^ w
            c = a & w
        else:
            u = a ^ w
            s = u ^ stored
            c = (a & w) | (u & stored)
        planes[k] = s
        w = c
        k += 1
    pend[k] = w


def _local_tree(p, g, base):
    ltp = [None] * 8
    lti = [None] * 8
    ltp_pend = [None] * 10
    lti_pend = [None] * 10
    for i2 in range(_SUBCHUNK):
        ohp = 1 << p[base + i2]
        ohg = 1 << g[base + i2]
        u = ohp ^ ohg
        both = ohp & ohg
        pl0 = ltp[0]
        if pl0 is None:
            ltp[0] = u
            carry = both
        else:
            ltp[0] = u ^ pl0
            carry = both | (u & pl0)
        _feed_single(ltp, ltp_pend, carry, 1)
        _feed_single(lti, lti_pend, both, 0)
    return (ltp, ltp_pend), (lti, lti_pend)


def _merge_into(ref, sidx, depth, planes, pend):
    carry = None
    for kk in range(depth):
        local = None
        if kk < len(planes) and planes[kk] is not None:
            local = planes[kk]
        if kk < len(pend) and pend[kk] is not None:
            assert local is None
            local = pend[kk]
        if local is None and carry is None:
            continue
        old = ref[sidx, kk]
        if local is not None and carry is not None:
            u = local ^ carry
            s = u ^ old
            c = (local & carry) | (u & old)
        else:
            w = local if local is not None else carry
            s = old ^ w
            c = old & w
        ref[sidx, kk] = s
        carry = c if kk < depth - 1 else None


def _fwiou_kernel(p_ref, g_ref, o_ref, tpl_ref, ipl_ref, *, n, block_rows):
    j = pl.program_id(0)

    @pl.when(j == 0)
    def _():
        tpl_ref[...] = jnp.zeros_like(tpl_ref)
        ipl_ref[...] = jnp.zeros_like(ipl_ref)

    chunk = _NSETS * _SUBCHUNK

    def chunk_body(ci, _):
        base = ci * chunk
        p = p_ref[pl.ds(base, chunk)]
        g = g_ref[pl.ds(base, chunk)]
        states = [_local_tree(p, g, s * _SUBCHUNK) for s in range(_NSETS)]
        for s, ((ltp, ltp_pend), (lti, lti_pend)) in enumerate(states):
            _merge_into(tpl_ref, s, _DT, ltp, ltp_pend)
            _merge_into(ipl_ref, s, _DI, lti, lti_pend)
        return 0

    jax.lax.fori_loop(0, block_rows // chunk, chunk_body, 0)

    @pl.when(j == _STEPS - 1)
    def _():
        def class_counts(ref, depth):
            accs = []
            for c in range(_N_CLASS):
                acc = None
                for s in range(_NSETS):
                    for k in range(depth):
                        b = ((ref[s, k] >> c) & 1) << k
                        acc = b if acc is None else acc + b
                accs.append(acc)
            x = jnp.stack(accs).astype(jnp.float32)
            return jnp.sum(jnp.sum(x, axis=2), axis=1)

        tl = class_counts(tpl_ref, _DT)
        il = class_counts(ipl_ref, _DI)
        weight = tl * (1.0 / (2.0 * n))
        iou = (il + _EPS) / ((tl - il) + _EPS)
        r = jnp.sum(weight * iou)
        o_ref[...] = jnp.full((1, _LANES), r, jnp.float32)


def kernel(class_pred, class_gt):
    n = class_pred.size
    rows = n // (_SUB * _LANES)
    block_rows = rows // _STEPS
    p = class_pred.reshape(rows, _SUB, _LANES)
    g = class_gt.reshape(rows, _SUB, _LANES)

    in_spec = pl.BlockSpec((block_rows, _SUB, _LANES), lambda j: (j, 0, 0))
    out = pl.pallas_call(
        functools.partial(_fwiou_kernel, n=float(n), block_rows=block_rows),
        grid=(_STEPS,),
        in_specs=[in_spec, in_spec],
        out_specs=pl.BlockSpec((1, _LANES), lambda j: (0, 0)),
        out_shape=jax.ShapeDtypeStruct((1, _LANES), jnp.float32),
        scratch_shapes=[
            pltpu.VMEM((_NSETS, _DT, _SUB, _LANES), jnp.int32),
            pltpu.VMEM((_NSETS, _DI, _SUB, _LANES), jnp.int32),
        ],
        compiler_params=pltpu.CompilerParams(
            dimension_semantics=("arbitrary",),
        ),
        name="fwiou_csa",
    )(p, g)
    return out[0, 0]
